# baseline (speedup 1.0000x reference)
import jax
import jax.numpy as jnp
from jax.experimental import pallas as pl
from jax.experimental.pallas import tpu as pltpu

N = 2
P2 = 64
W2 = 64
DIM = 1024
QK = 1024
KV = 2048
TOPK = 4
SCALE = QK ** -0.5
PB = 16


def _dot(a, b, precision=None):
    return jax.lax.dot_general(a, b, (((1,), (0,)), ((), ())),
                               preferred_element_type=jnp.float32,
                               precision=precision)


def _dot_t(a, b):
    return jax.lax.dot_general(a, b, (((1,), (1,)), ((), ())),
                               preferred_element_type=jnp.float32)


def _qkv_kernel(x_ref, w_ref, b_ref, q_ref, kv_ref, idx_ref, wgt_ref,
                qw_s, kw_s):
    g = pl.program_id(0)
    gb = P2 // PB
    x = x_ref[...]
    qkv = _dot(x, w_ref[...]) + b_ref[...]
    q = qkv[:, :QK]
    kv = qkv[:, QK:]
    q_ref[...] = q.astype(jnp.bfloat16)
    kv_ref[...] = kv.astype(jnp.bfloat16)
    r = (g % gb) * PB
    qw_s[pl.ds(r, PB), :] = jnp.mean(q.reshape(PB, W2, QK), axis=1)
    kw_s[pl.ds(r, PB), :] = jnp.mean(kv[:, :QK].reshape(PB, W2, QK), axis=1)

    @pl.when(g % gb == gb - 1)
    def _():
        logit = _dot_t(qw_s[...] * SCALE, kw_s[...])
        col = jax.lax.broadcasted_iota(jnp.int32, (P2, P2), 1)
        lane = jax.lax.broadcasted_iota(jnp.int32, (P2, 128), 1)
        idx_out = jnp.zeros((P2, 128), jnp.int32)
        val_out = jnp.zeros((P2, 128), jnp.float32)
        cur = logit
        for t in range(TOPK):
            m = jnp.max(cur, axis=-1, keepdims=True)
            a = jnp.min(jnp.where(cur == m, col, P2), axis=-1, keepdims=True)
            idx_out = jnp.where(lane == t, a, idx_out)
            val_out = jnp.where(lane == t, m, val_out)
            cur = jnp.where(col == a, -jnp.inf, cur)
        e = jnp.where(lane < TOPK, jnp.exp(val_out - val_out[:, :1]), 0.0)
        s = jnp.sum(e, axis=-1, keepdims=True)
        idx_ref[...] = idx_out
        wgt_ref[...] = e / s


G = 16


def _attn_kernel(ridx_ref, q_ref, kv_ref, rw_ref, wo_ref, bo_ref, o_ref):
    b = pl.program_id(0)
    jj = pl.program_id(1)
    lane = jax.lax.broadcasted_iota(jnp.int32, (1, 128), 1)
    wts_all, kvt_all, l_all = [], [], []
    for w in range(G):
        row = b * P2 + jj * G + w
        q = q_ref[w * W2:(w + 1) * W2, :]
        rww = rw_ref[w:w + 1, :]
        wts = [jnp.sum(jnp.where(lane == t, rww, 0.0)) for t in range(TOPK)]
        kvt = [kv_ref[0, pl.ds(ridx_ref[row, t] * W2, W2), :]
               for t in range(TOPK)]
        ls = [_dot_t(q, kvt[t][:, :QK]) * (wts[t] * SCALE)
              for t in range(TOPK)]
        wts_all.append(wts)
        kvt_all.append(kvt)
        l_all.append(jnp.concatenate(ls, axis=1))
    p_all = []
    for w in range(G):
        l = l_all[w]
        m = jnp.max(l, axis=-1, keepdims=True)
        p = jnp.exp(l - m)
        s = jnp.sum(p, axis=-1, keepdims=True)
        p_all.append((p, s))
    outs = []
    for w in range(G):
        p, s = p_all[w]
        wts, kvt = wts_all[w], kvt_all[w]
        acc = _dot((p[:, :W2] * wts[0]).astype(jnp.bfloat16), kvt[0][:, QK:])
        for t in range(1, TOPK):
            pt = (p[:, t * W2:(t + 1) * W2] * wts[t]).astype(jnp.bfloat16)
            acc += _dot(pt, kvt[t][:, QK:])
        outs.append((acc / s).astype(jnp.bfloat16))
    o_ref[...] = _dot(jnp.concatenate(outs, axis=0), wo_ref[...]) + bo_ref[...]


def kernel(x, W_qkv, b_qkv, W_o, b_o):
    n, p2, w2, dim = x.shape
    rows = n * p2 * w2
    x2 = x.reshape(rows, dim)
    b2 = b_qkv.reshape(1, 2 * QK + DIM)

    gb = P2 // PB
    q2, kv2, r_idx, r_wgt = pl.pallas_call(
        _qkv_kernel,
        grid=(rows // (PB * W2),),
        in_specs=[
            pl.BlockSpec((PB * W2, DIM), lambda g: (g, 0)),
            pl.BlockSpec((DIM, 2 * QK + DIM), lambda g: (0, 0)),
            pl.BlockSpec((1, 2 * QK + DIM), lambda g: (0, 0)),
        ],
        out_specs=[
            pl.BlockSpec((PB * W2, QK), lambda g: (g, 0)),
            pl.BlockSpec((PB * W2, KV), lambda g: (g, 0)),
            pl.BlockSpec((P2, 128), lambda g: (g // gb, 0)),
            pl.BlockSpec((P2, 128), lambda g: (g // gb, 0)),
        ],
        out_shape=[
            jax.ShapeDtypeStruct((rows, QK), jnp.bfloat16),
            jax.ShapeDtypeStruct((rows, KV), jnp.bfloat16),
            jax.ShapeDtypeStruct((n * p2, 128), jnp.int32),
            jax.ShapeDtypeStruct((n * p2, 128), jnp.float32),
        ],
        scratch_shapes=[
            pltpu.VMEM((P2, QK), jnp.float32),
            pltpu.VMEM((P2, QK), jnp.float32),
        ],
        compiler_params=pltpu.CompilerParams(
            dimension_semantics=("arbitrary",)),
    )(x2, W_qkv, b2)

    bo2 = b_o.reshape(1, DIM)

    out = pl.pallas_call(
        _attn_kernel,
        grid_spec=pltpu.PrefetchScalarGridSpec(
            num_scalar_prefetch=1,
            grid=(n, p2 // G),
            in_specs=[
                pl.BlockSpec((G * W2, QK),
                             lambda b, jj, ridx: (b * (P2 // G) + jj, 0)),
                pl.BlockSpec((1, P2 * W2, KV), lambda b, jj, ridx: (b, 0, 0)),
                pl.BlockSpec((G, 128),
                             lambda b, jj, ridx: (b * (P2 // G) + jj, 0)),
                pl.BlockSpec((DIM, DIM), lambda b, jj, ridx: (0, 0)),
                pl.BlockSpec((1, DIM), lambda b, jj, ridx: (0, 0)),
            ],
            out_specs=pl.BlockSpec(
                (G * W2, DIM), lambda b, jj, ridx: (b * (P2 // G) + jj, 0)),
        ),
        out_shape=jax.ShapeDtypeStruct((rows, DIM), jnp.float32),
        compiler_params=pltpu.CompilerParams(
            dimension_semantics=("parallel", "parallel"),
            vmem_limit_bytes=100 * 1024 * 1024),
    )(r_idx, q2, kv2.reshape(n, p2 * w2, KV), r_wgt,
      W_o.astype(jnp.bfloat16), bo2)

    return out.reshape(n, p2, w2, dim)
